# baseline (speedup 1.0000x reference)
import functools

import jax
import jax.numpy as jnp
from jax import lax
from jax.experimental import pallas as pl
from jax.experimental.pallas import tpu as pltpu
from jax.experimental.pallas import tpu_sc as plsc

MASK_VALUE = 1000000000.0
NUM_CORES = 2
NUM_SUBCORES = 16
NUM_WORKERS = NUM_CORES * NUM_SUBCORES
LANES = 16
VOCAB = 14
D = 128
CHUNK = 32
NBUF = 4
PREF = 2


@functools.lru_cache(maxsize=None)
def _make_kernel(n_rows):
    rows_per_w = n_rows // NUM_WORKERS
    slots = rows_per_w // CHUNK
    assert n_rows % (NUM_WORKERS * CHUNK) == 0 and slots % NBUF == 0 and slots > 2 * NBUF
    mesh = plsc.VectorSubcoreMesh(core_axis_name="c", subcore_axis_name="s")

    @functools.partial(
        pl.kernel,
        out_type=jax.ShapeDtypeStruct((n_rows // CHUNK, CHUNK, D), jnp.float32),
        mesh=mesh,
        compiler_params=pltpu.CompilerParams(needs_layout_passes=False),
        scratch_types=[
            pltpu.VMEM((slots, CHUNK), jnp.int32),
            pltpu.VMEM((VOCAB * D,), jnp.float32),
            pltpu.VMEM_SHARED((VOCAB, D), jnp.float32),
        ]
        + [pltpu.VMEM((CHUNK, D), jnp.float32)] * NBUF
        + [pltpu.SemaphoreType.DMA] * NBUF
        + [pltpu.SemaphoreType.DMA] * NBUF,
    )
    def board_embed(table_hbm, idx_hbm, out_hbm, idx_v, tmask_v, tab_sh, *rest):
        bufs = rest[:NBUF]
        gsems = rest[NBUF : 2 * NBUF]
        ssems = rest[2 * NBUF :]
        sid = lax.axis_index("s")
        wid = sid * NUM_CORES + lax.axis_index("c")
        pltpu.sync_copy(idx_hbm.at[wid], idx_v)

        @pl.when(sid == 0)
        def _stage():
            pltpu.sync_copy(table_hbm, tmask_v)
            for v in range(VOCAB):
                chunks = [tmask_v[pl.ds(v * D + k * LANES, LANES)] for k in range(D // LANES)]
                ne = jnp.where(chunks[0] != MASK_VALUE, jnp.float32(1.0), jnp.float32(0.0))
                for c in chunks[1:]:
                    ne = jnp.maximum(
                        ne, jnp.where(c != MASK_VALUE, jnp.float32(1.0), jnp.float32(0.0))
                    )
                keep = jnp.max(ne)
                for k, c in enumerate(chunks):
                    tmask_v[pl.ds(v * D + k * LANES, LANES)] = c * keep
            for v in range(VOCAB):
                pltpu.sync_copy(tmask_v.at[pl.ds(v * D, D)], tab_sh.at[v])

        plsc.subcore_barrier()

        def issue_gather(slot, b):
            pltpu.async_copy(tab_sh.at[idx_v.at[slot]], bufs[b], gsems[b])

        def drain_gather(slot, b):
            pltpu.make_async_copy(tab_sh.at[idx_v.at[slot]], bufs[b], gsems[b]).wait()

        def issue_store(slot, b):
            pltpu.async_copy(bufs[b], out_hbm.at[wid * slots + slot], ssems[b])

        def wait_store(slot, b):
            pltpu.make_async_copy(bufs[b], out_hbm.at[wid * slots + slot], ssems[b]).wait()

        def step(slot, j, first, last):
            drain_gather(slot, j)
            issue_store(slot, j)
            bp = (j + PREF) % NBUF
            if not first:
                wait_store(slot - PREF, bp)
            if not last:
                issue_gather(slot + PREF, bp)

        for j in range(PREF):
            issue_gather(j, j)
        for j in range(NBUF):
            step(j, j, first=(j < PREF), last=False)

        def ring_round(i, carry):
            for j in range(NBUF):
                slot = (i + 1) * NBUF + j
                step(slot, j, first=False, last=False)
            return carry

        lax.fori_loop(0, slots // NBUF - 2, ring_round, 0)

        for j in range(NBUF):
            slot = slots - NBUF + j
            step(slot, j, first=False, last=(j >= NBUF - PREF))
        for j in range(NBUF - PREF, NBUF):
            wait_store(slots - NBUF + j, j)

    return board_embed


def kernel(inputs, table):
    b = inputs.shape[0]
    n_rows = b * 64
    flat = inputs.reshape(NUM_WORKERS, n_rows // (NUM_WORKERS * CHUNK), CHUNK)
    out = _make_kernel(n_rows)(table.reshape(-1), flat)
    return out.reshape(b, 64, D)


# speedup vs baseline: 1.3021x; 1.3021x over previous
import functools

import jax
import jax.numpy as jnp
from jax import lax
from jax.experimental import pallas as pl
from jax.experimental.pallas import tpu as pltpu
from jax.experimental.pallas import tpu_sc as plsc

MASK_VALUE = 1000000000.0
NUM_CORES = 2
NUM_SUBCORES = 16
NUM_WORKERS = NUM_CORES * NUM_SUBCORES
LANES = 16
VOCAB = 14
D = 128
CHUNK = 64
NBUF = 8
PREF = 4


@functools.lru_cache(maxsize=None)
def _make_kernel(n_rows):
    rows_per_w = n_rows // NUM_WORKERS
    slots = rows_per_w // CHUNK
    assert n_rows % (NUM_WORKERS * CHUNK) == 0 and slots % NBUF == 0 and slots > 2 * NBUF
    mesh = plsc.VectorSubcoreMesh(core_axis_name="c", subcore_axis_name="s")

    @functools.partial(
        pl.kernel,
        out_type=jax.ShapeDtypeStruct((n_rows // CHUNK, CHUNK, D), jnp.float32),
        mesh=mesh,
        compiler_params=pltpu.CompilerParams(needs_layout_passes=False),
        scratch_types=[
            pltpu.VMEM((slots, CHUNK), jnp.int32),
            pltpu.VMEM((VOCAB * D,), jnp.float32),
            pltpu.VMEM_SHARED((VOCAB, D), jnp.float32),
        ]
        + [pltpu.VMEM((CHUNK, D), jnp.float32)] * NBUF
        + [pltpu.SemaphoreType.DMA] * NBUF
        + [pltpu.SemaphoreType.DMA] * NBUF,
    )
    def board_embed(table_hbm, idx_hbm, out_hbm, idx_v, tmask_v, tab_sh, *rest):
        bufs = rest[:NBUF]
        gsems = rest[NBUF : 2 * NBUF]
        ssems = rest[2 * NBUF :]
        sid = lax.axis_index("s")
        wid = sid * NUM_CORES + lax.axis_index("c")
        pltpu.sync_copy(idx_hbm.at[wid], idx_v)

        @pl.when(sid == 0)
        def _stage():
            pltpu.sync_copy(table_hbm, tmask_v)
            for v in range(VOCAB):
                chunks = [tmask_v[pl.ds(v * D + k * LANES, LANES)] for k in range(D // LANES)]
                ne = jnp.where(chunks[0] != MASK_VALUE, jnp.float32(1.0), jnp.float32(0.0))
                for c in chunks[1:]:
                    ne = jnp.maximum(
                        ne, jnp.where(c != MASK_VALUE, jnp.float32(1.0), jnp.float32(0.0))
                    )
                keep = jnp.max(ne)
                for k, c in enumerate(chunks):
                    tmask_v[pl.ds(v * D + k * LANES, LANES)] = c * keep
            for v in range(VOCAB):
                pltpu.sync_copy(tmask_v.at[pl.ds(v * D, D)], tab_sh.at[v])

        plsc.subcore_barrier()

        def issue_gather(slot, b):
            pltpu.async_copy(tab_sh.at[idx_v.at[slot]], bufs[b], gsems[b])

        def drain_gather(slot, b):
            pltpu.make_async_copy(tab_sh.at[idx_v.at[slot]], bufs[b], gsems[b]).wait()

        def issue_store(slot, b):
            pltpu.async_copy(bufs[b], out_hbm.at[wid * slots + slot], ssems[b])

        def wait_store(slot, b):
            pltpu.make_async_copy(bufs[b], out_hbm.at[wid * slots + slot], ssems[b]).wait()

        def step(slot, j, first, last):
            drain_gather(slot, j)
            issue_store(slot, j)
            bp = (j + PREF) % NBUF
            if not first:
                wait_store(slot - PREF, bp)
            if not last:
                issue_gather(slot + PREF, bp)

        for j in range(PREF):
            issue_gather(j, j)
        for j in range(NBUF):
            step(j, j, first=(j < PREF), last=False)

        def ring_round(i, carry):
            for j in range(NBUF):
                slot = (i + 1) * NBUF + j
                step(slot, j, first=False, last=False)
            return carry

        lax.fori_loop(0, slots // NBUF - 2, ring_round, 0)

        for j in range(NBUF):
            slot = slots - NBUF + j
            step(slot, j, first=False, last=(j >= NBUF - PREF))
        for j in range(NBUF - PREF, NBUF):
            wait_store(slots - NBUF + j, j)

    return board_embed


def kernel(inputs, table):
    b = inputs.shape[0]
    n_rows = b * 64
    flat = inputs.reshape(NUM_WORKERS, n_rows // (NUM_WORKERS * CHUNK), CHUNK)
    out = _make_kernel(n_rows)(table.reshape(-1), flat)
    return out.reshape(b, 64, D)


# speedup vs baseline: 1.3229x; 1.0160x over previous
import functools

import jax
import jax.numpy as jnp
from jax import lax
from jax.experimental import pallas as pl
from jax.experimental.pallas import tpu as pltpu
from jax.experimental.pallas import tpu_sc as plsc

MASK_VALUE = 1000000000.0
NUM_CORES = 2
NUM_SUBCORES = 16
NUM_WORKERS = NUM_CORES * NUM_SUBCORES
LANES = 16
VOCAB = 14
D = 128
CHUNK = 64
NBUF = 8
PREF = 4


@functools.lru_cache(maxsize=None)
def _make_kernel(n_rows):
    rows_per_w = n_rows // NUM_WORKERS
    slots = rows_per_w // CHUNK
    assert n_rows % (NUM_WORKERS * CHUNK) == 0 and slots % NBUF == 0 and slots > 2 * NBUF
    mesh = plsc.VectorSubcoreMesh(core_axis_name="c", subcore_axis_name="s")

    @functools.partial(
        pl.kernel,
        out_type=jax.ShapeDtypeStruct((n_rows // CHUNK, CHUNK, D), jnp.float32),
        mesh=mesh,
        compiler_params=pltpu.CompilerParams(needs_layout_passes=False),
        scratch_types=[
            pltpu.VMEM((slots, CHUNK), jnp.int32),
            pltpu.VMEM((VOCAB * D,), jnp.float32),
            pltpu.VMEM_SHARED((VOCAB, D), jnp.float32),
        ]
        + [pltpu.VMEM((CHUNK, D), jnp.float32)] * NBUF
        + [pltpu.SemaphoreType.DMA] * NBUF
        + [pltpu.SemaphoreType.DMA] * NBUF
        + [pltpu.SemaphoreType.DMA],
    )
    def board_embed(table_hbm, idx_hbm, out_hbm, idx_v, tmask_v, tab_sh, *rest):
        bufs = rest[:NBUF]
        gsems = rest[NBUF : 2 * NBUF]
        ssems = rest[2 * NBUF : 3 * NBUF]
        isem = rest[3 * NBUF]
        sid = lax.axis_index("s")
        wid = sid * NUM_CORES + lax.axis_index("c")
        idx_copy = pltpu.async_copy(idx_hbm.at[wid], idx_v, isem)

        @pl.when(sid == 0)
        def _stage():
            pltpu.sync_copy(table_hbm, tmask_v)
            for v in range(VOCAB):
                chunks = [tmask_v[pl.ds(v * D + k * LANES, LANES)] for k in range(D // LANES)]
                ne = jnp.where(chunks[0] != MASK_VALUE, jnp.float32(1.0), jnp.float32(0.0))
                for c in chunks[1:]:
                    ne = jnp.maximum(
                        ne, jnp.where(c != MASK_VALUE, jnp.float32(1.0), jnp.float32(0.0))
                    )
                keep = jnp.max(ne)
                for k, c in enumerate(chunks):
                    tmask_v[pl.ds(v * D + k * LANES, LANES)] = c * keep
            for v in range(VOCAB):
                pltpu.sync_copy(tmask_v.at[pl.ds(v * D, D)], tab_sh.at[v])

        plsc.subcore_barrier()
        idx_copy.wait()

        def issue_gather(slot, b):
            pltpu.async_copy(tab_sh.at[idx_v.at[slot]], bufs[b], gsems[b])

        def drain_gather(slot, b):
            pltpu.make_async_copy(tab_sh.at[idx_v.at[slot]], bufs[b], gsems[b]).wait()

        def issue_store(slot, b):
            pltpu.async_copy(bufs[b], out_hbm.at[wid * slots + slot], ssems[b])

        def wait_store(slot, b):
            pltpu.make_async_copy(bufs[b], out_hbm.at[wid * slots + slot], ssems[b]).wait()

        def step(slot, j, first, last):
            drain_gather(slot, j)
            issue_store(slot, j)
            bp = (j + PREF) % NBUF
            if not first:
                wait_store(slot - PREF, bp)
            if not last:
                issue_gather(slot + PREF, bp)

        for j in range(PREF):
            issue_gather(j, j)
        for j in range(NBUF):
            step(j, j, first=(j < PREF), last=False)

        def ring_round(i, carry):
            for j in range(NBUF):
                slot = (i + 1) * NBUF + j
                step(slot, j, first=False, last=False)
            return carry

        lax.fori_loop(0, slots // NBUF - 2, ring_round, 0)

        for j in range(NBUF):
            slot = slots - NBUF + j
            step(slot, j, first=False, last=(j >= NBUF - PREF))
        for j in range(NBUF - PREF, NBUF):
            wait_store(slots - NBUF + j, j)

    return board_embed


def kernel(inputs, table):
    b = inputs.shape[0]
    n_rows = b * 64
    flat = inputs.reshape(NUM_WORKERS, n_rows // (NUM_WORKERS * CHUNK), CHUNK)
    out = _make_kernel(n_rows)(table.reshape(-1), flat)
    return out.reshape(b, 64, D)


# speedup vs baseline: 1.3408x; 1.0136x over previous
import functools

import jax
import jax.numpy as jnp
from jax import lax
from jax.experimental import pallas as pl
from jax.experimental.pallas import tpu as pltpu
from jax.experimental.pallas import tpu_sc as plsc

MASK_VALUE = 1000000000.0
NUM_CORES = 2
NUM_SUBCORES = 16
NUM_WORKERS = NUM_CORES * NUM_SUBCORES
LANES = 16
VOCAB = 14
D = 128
CHUNK = 64
NBUF = 8
PREF = 4


@functools.lru_cache(maxsize=None)
def _make_kernel(n_rows):
    rows_per_w = n_rows // NUM_WORKERS
    slots = rows_per_w // CHUNK
    assert n_rows % (NUM_WORKERS * CHUNK) == 0 and slots % NBUF == 0 and slots > 2 * NBUF
    mesh = plsc.VectorSubcoreMesh(core_axis_name="c", subcore_axis_name="s")

    @functools.partial(
        pl.kernel,
        out_type=jax.ShapeDtypeStruct((n_rows // CHUNK, CHUNK, D), jnp.float32),
        mesh=mesh,
        compiler_params=pltpu.CompilerParams(needs_layout_passes=False),
        scratch_types=[
            pltpu.VMEM((slots, CHUNK), jnp.int32),
            pltpu.VMEM((D,), jnp.float32),
            pltpu.VMEM_SHARED((VOCAB, D), jnp.float32),
        ]
        + [pltpu.VMEM((CHUNK, D), jnp.float32)] * NBUF
        + [pltpu.SemaphoreType.DMA] * NBUF
        + [pltpu.SemaphoreType.DMA] * NBUF
        + [pltpu.SemaphoreType.DMA],
    )
    def board_embed(table_hbm, idx_hbm, out_hbm, idx_v, tmask_v, tab_sh, *rest):
        bufs = rest[:NBUF]
        gsems = rest[NBUF : 2 * NBUF]
        ssems = rest[2 * NBUF : 3 * NBUF]
        isem = rest[3 * NBUF]
        sid = lax.axis_index("s")
        wid = sid * NUM_CORES + lax.axis_index("c")
        idx_copy = pltpu.async_copy(idx_hbm.at[wid], idx_v, isem)

        @pl.when(sid < VOCAB)
        def _stage():
            pltpu.sync_copy(table_hbm.at[pl.ds(sid * D, D)], tmask_v)
            chunks = [tmask_v[pl.ds(k * LANES, LANES)] for k in range(D // LANES)]
            ne = jnp.where(chunks[0] != MASK_VALUE, jnp.float32(1.0), jnp.float32(0.0))
            for c in chunks[1:]:
                ne = jnp.maximum(
                    ne, jnp.where(c != MASK_VALUE, jnp.float32(1.0), jnp.float32(0.0))
                )
            keep = jnp.max(ne)
            for k, c in enumerate(chunks):
                tmask_v[pl.ds(k * LANES, LANES)] = c * keep
            pltpu.sync_copy(tmask_v, tab_sh.at[sid])

        plsc.subcore_barrier()
        idx_copy.wait()

        def issue_gather(slot, b):
            pltpu.async_copy(tab_sh.at[idx_v.at[slot]], bufs[b], gsems[b])

        def drain_gather(slot, b):
            pltpu.make_async_copy(tab_sh.at[idx_v.at[slot]], bufs[b], gsems[b]).wait()

        def issue_store(slot, b):
            pltpu.async_copy(bufs[b], out_hbm.at[wid * slots + slot], ssems[b])

        def wait_store(slot, b):
            pltpu.make_async_copy(bufs[b], out_hbm.at[wid * slots + slot], ssems[b]).wait()

        def step(slot, j, first, last):
            drain_gather(slot, j)
            issue_store(slot, j)
            bp = (j + PREF) % NBUF
            if not first:
                wait_store(slot - PREF, bp)
            if not last:
                issue_gather(slot + PREF, bp)

        for j in range(PREF):
            issue_gather(j, j)
        for j in range(NBUF):
            step(j, j, first=(j < PREF), last=False)

        def ring_round(i, carry):
            for j in range(NBUF):
                slot = (i + 1) * NBUF + j
                step(slot, j, first=False, last=False)
            return carry

        lax.fori_loop(0, slots // NBUF - 2, ring_round, 0)

        for j in range(NBUF):
            slot = slots - NBUF + j
            step(slot, j, first=False, last=(j >= NBUF - PREF))
        for j in range(NBUF - PREF, NBUF):
            wait_store(slots - NBUF + j, j)

    return board_embed


def kernel(inputs, table):
    b = inputs.shape[0]
    n_rows = b * 64
    flat = inputs.reshape(NUM_WORKERS, n_rows // (NUM_WORKERS * CHUNK), CHUNK)
    out = _make_kernel(n_rows)(table.reshape(-1), flat)
    return out.reshape(b, 64, D)
